# baseline (speedup 1.0000x reference)
import functools

import jax
import jax.numpy as jnp
from jax import lax
from jax.experimental import pallas as pl
from jax.experimental.pallas import tpu as pltpu
from jax.experimental.pallas import tpu_sc as plsc

N_NODES = 10000
D_IN = 128
HIDDEN = 128
LABEL_DIM = 40
LABEL_PAD = 48

NC = 2
NS = 16
K = 128


def _broadcast_lane(v16, lane):
    return lax.gather(
        v16, jnp.full((16, 1), lane, jnp.int32),
        lax.GatherDimensionNumbers(
            offset_dims=(), collapsed_slice_dims=(0,), start_index_map=(0,)),
        (1,), mode=lax.GatherScatterMode.PROMISE_IN_BOUNDS)


def _make_sc_spmm(n_pad, d, e_pad, split, m):
    d_io = d // 2 if split else d
    M = m * 128
    n_workers = NS if split else NC * NS
    per_w = e_pad // n_workers
    n_chunks = per_w // M
    assert n_chunks % 2 == 0 and per_w % M == 0
    rows_per_tile = n_pad // NS
    zfull, zrem = divmod(rows_per_tile, M)
    mesh = plsc.VectorSubcoreMesh(core_axis_name="c", subcore_axis_name="s")

    @functools.partial(
        pl.kernel,
        out_type=jax.ShapeDtypeStruct((NC, n_pad, d_io), jnp.float32),
        mesh=mesh,
        scratch_types=[
            pltpu.VMEM((2, M), jnp.int32),
            pltpu.VMEM((2, M), jnp.int32),
            pltpu.VMEM((2, M), jnp.float32),
            pltpu.VMEM((2, M, d_io), jnp.float32),
            pltpu.VMEM_SHARED((n_pad, d_io), jnp.float32),
            pltpu.SemaphoreType.DMA,
            pltpu.SemaphoreType.DMA,
            pltpu.SemaphoreType.DMA,
            pltpu.SemaphoreType.DMA,
        ],
        compiler_params=pltpu.CompilerParams(use_tc_tiling_on_sc=False),
    )
    def spmm(src_hbm, dst_hbm, w_hbm, h_hbm, out_hbm,
             sidx_v, didx_v, w_v, rows_v, acc_sh, g0, g1, s0, s1):
        c = lax.axis_index("c")
        s = lax.axis_index("s")
        wid = s if split else s * NC + c
        ebase = wid * n_chunks * M
        row0 = s * rows_per_tile
        gsem = (g0, g1)
        ssem = (s0, s1)

        def idx_fetch(k, b):
            pltpu.sync_copy(src_hbm.at[pl.ds(ebase + k * M, M)], sidx_v.at[b])
            pltpu.sync_copy(dst_hbm.at[pl.ds(ebase + k * M, M)], didx_v.at[b])
            pltpu.sync_copy(w_hbm.at[pl.ds(ebase + k * M, M)], w_v.at[b])
            if split:

                def sxf(gg, inner):
                    sl = pl.ds(gg * 16, 16)
                    sidx_v[b, sl] = sidx_v[b, sl] * 2 + c
                    return inner
                lax.fori_loop(0, M // 16, sxf, 0)

        def gather_start(b):
            pltpu.async_copy(h_hbm.at[sidx_v.at[b]], rows_v.at[b], gsem[b])

        def gather_wait(b):
            pltpu.make_async_copy(
                h_hbm.at[sidx_v.at[b]], rows_v.at[b], gsem[b]).wait()

        def scatter_start(b):
            pltpu.async_copy(
                rows_v.at[b], acc_sh.at[didx_v.at[b]], ssem[b], add=True)

        def scatter_wait(b):
            pltpu.make_async_copy(
                rows_v.at[b], acc_sh.at[didx_v.at[b]], ssem[b]).wait()

        def scale(b):
            def edge(e, inner):
                g = e // 16
                lane = e - g * 16
                w16 = w_v[b, pl.ds(g * 16, 16)]
                wb = _broadcast_lane(w16, lane)
                for j in range(d_io // 16):
                    sl = pl.ds(j * 16, 16)
                    rows_v[b, e, sl] = rows_v[b, e, sl] * wb
                return inner
            lax.fori_loop(0, M, edge, 0)

        def zrow(i, carry):
            for j in range(d_io // 16):
                rows_v[0, i, pl.ds(j * 16, 16)] = jnp.zeros((16,), jnp.float32)
            return carry
        lax.fori_loop(0, M, zrow, 0)
        for q in range(zfull):
            pltpu.sync_copy(rows_v.at[0], acc_sh.at[pl.ds(row0 + q * M, M)])
        if zrem:
            pltpu.sync_copy(rows_v.at[0, pl.ds(0, zrem)],
                            acc_sh.at[pl.ds(row0 + zfull * M, zrem)])
        plsc.subcore_barrier()

        idx_fetch(0, 0)
        gather_start(0)

        def step(k, b):
            nb = 1 - b
            gather_wait(b)
            scale(b)
            scatter_start(b)
            if b == 1:
                scatter_wait(nb)
            else:

                @pl.when(k >= 1)
                def _():
                    scatter_wait(nb)

            @pl.when(k + 1 < n_chunks)
            def _():
                idx_fetch(k + 1, nb)
                gather_start(nb)

        def pair(k2, carry):
            step(k2 * 2, 0)
            step(k2 * 2 + 1, 1)
            return carry
        lax.fori_loop(0, n_chunks // 2, pair, 0)
        scatter_wait(1)
        plsc.subcore_barrier()

        pltpu.sync_copy(acc_sh.at[pl.ds(row0, rows_per_tile)],
                        out_hbm.at[c, pl.ds(row0, rows_per_tile)])

    return spmm


def _mm_body(x_ref, w_ref, o_ref):
    o_ref[...] = jnp.dot(x_ref[...], w_ref[...],
                         preferred_element_type=jnp.float32)


def _fuse1_body(p_ref, b_ref, w_ref, o_ref):
    h = jnp.maximum(p_ref[0] + p_ref[1] + b_ref[...], 0.0)
    o_ref[...] = jnp.dot(h, w_ref[...], preferred_element_type=jnp.float32)


def _fuse2_body(p_ref, b_ref, o_ref):
    s = p_ref[0] + p_ref[1] + b_ref[...]
    logits = s[:, :LABEL_DIM]
    m = jnp.max(logits, axis=1, keepdims=True)
    z = logits - m
    lse = jnp.log(jnp.sum(jnp.exp(z), axis=1, keepdims=True))
    o_ref[...] = z - lse


def kernel(x, edge_index, edge_weight, W1, b1, W2, b2):
    n = x.shape[0]
    e = edge_weight.shape[0]
    chunk_span = NC * NS * K * 2
    e_pad = ((e + chunk_span - 1) // chunk_span) * chunk_span
    row_span = NS * 8
    n_pad = ((n + row_span - 1) // row_span) * row_span

    src = edge_index[0].astype(jnp.int32)
    dst = edge_index[1].astype(jnp.int32)
    pad = e_pad - e
    if pad:
        src = jnp.pad(src, (0, pad))
        dst = jnp.pad(dst, (0, pad))
        edge_weight = jnp.pad(edge_weight, (0, pad))
    srcg = src
    dstg = dst
    wg = edge_weight

    w2p = jnp.pad(W2, ((0, 0), (0, LABEL_PAD - LABEL_DIM)))
    b1r = b1.reshape(1, HIDDEN)
    b2r = jnp.pad(b2, (0, LABEL_PAD - LABEL_DIM)).reshape(1, LABEL_PAD)

    h1 = pl.pallas_call(
        _mm_body,
        out_shape=jax.ShapeDtypeStruct((n, HIDDEN), jnp.float32),
    )(x, W1)

    spmm1 = _make_sc_spmm(n_pad, HIDDEN, e_pad, split=False, m=1)
    p1 = spmm1(srcg, dstg, wg, h1)

    h2 = pl.pallas_call(
        _fuse1_body,
        out_shape=jax.ShapeDtypeStruct((n_pad, LABEL_PAD), jnp.float32),
    )(p1, b1r, w2p)

    spmm2 = _make_sc_spmm(n_pad, LABEL_PAD, e_pad, split=False, m=1)
    p2 = spmm2(srcg, dstg, wg, h2)

    out = pl.pallas_call(
        _fuse2_body,
        out_shape=jax.ShapeDtypeStruct((n_pad, LABEL_DIM), jnp.float32),
    )(p2, b2r)
    return out[:n]


# speedup vs baseline: 1.4359x; 1.4359x over previous
import functools

import jax
import jax.numpy as jnp
from jax import lax
from jax.experimental import pallas as pl
from jax.experimental.pallas import tpu as pltpu
from jax.experimental.pallas import tpu_sc as plsc

N_NODES = 10000
D_IN = 128
HIDDEN = 128
LABEL_DIM = 40
LABEL_PAD = 48

NC = 2
NS = 16
K = 128


def _broadcast_lane(v16, lane):
    return lax.gather(
        v16, jnp.full((16, 1), lane, jnp.int32),
        lax.GatherDimensionNumbers(
            offset_dims=(), collapsed_slice_dims=(0,), start_index_map=(0,)),
        (1,), mode=lax.GatherScatterMode.PROMISE_IN_BOUNDS)


def _make_sc_spmm(n_pad, d, e_pad, split, m):
    d_io = d // 2 if split else d
    M = m * 128
    n_workers = NS if split else NC * NS
    per_w = e_pad // n_workers
    n_chunks = per_w // M
    assert n_chunks % 2 == 0 and per_w % M == 0
    rows_per_tile = n_pad // NS
    zfull, zrem = divmod(rows_per_tile, M)
    mesh = plsc.VectorSubcoreMesh(core_axis_name="c", subcore_axis_name="s")

    @functools.partial(
        pl.kernel,
        out_type=jax.ShapeDtypeStruct((NC, n_pad, d_io), jnp.float32),
        mesh=mesh,
        scratch_types=[
            pltpu.VMEM((2, M), jnp.int32),
            pltpu.VMEM((2, M), jnp.int32),
            pltpu.VMEM((2, M), jnp.float32),
            pltpu.VMEM((2, M, d_io), jnp.float32),
            pltpu.VMEM_SHARED((n_pad, d_io), jnp.float32),
            pltpu.SemaphoreType.DMA,
            pltpu.SemaphoreType.DMA,
            pltpu.SemaphoreType.DMA,
            pltpu.SemaphoreType.DMA,
        ],
        compiler_params=pltpu.CompilerParams(use_tc_tiling_on_sc=False),
    )
    def spmm(src_hbm, dst_hbm, w_hbm, h_hbm, out_hbm,
             sidx_v, didx_v, w_v, rows_v, acc_sh, g0, g1, s0, s1):
        c = lax.axis_index("c")
        s = lax.axis_index("s")
        wid = s if split else s * NC + c
        ebase = wid * n_chunks * M
        row0 = s * rows_per_tile
        gsem = (g0, g1)
        ssem = (s0, s1)

        def idx_fetch(k, b):
            pltpu.sync_copy(src_hbm.at[pl.ds(ebase + k * M, M)], sidx_v.at[b])
            pltpu.sync_copy(dst_hbm.at[pl.ds(ebase + k * M, M)], didx_v.at[b])
            pltpu.sync_copy(w_hbm.at[pl.ds(ebase + k * M, M)], w_v.at[b])
            if split:

                def sxf(gg, inner):
                    sl = pl.ds(gg * 16, 16)
                    sidx_v[b, sl] = sidx_v[b, sl] * 2 + c
                    return inner
                lax.fori_loop(0, M // 16, sxf, 0)

        def gather_start(b):
            pltpu.async_copy(h_hbm.at[sidx_v.at[b]], rows_v.at[b], gsem[b])

        def gather_wait(b):
            pltpu.make_async_copy(
                h_hbm.at[sidx_v.at[b]], rows_v.at[b], gsem[b]).wait()

        def scatter_start(b):
            pltpu.async_copy(
                rows_v.at[b], acc_sh.at[didx_v.at[b]], ssem[b], add=True)

        def scatter_wait(b):
            pltpu.make_async_copy(
                rows_v.at[b], acc_sh.at[didx_v.at[b]], ssem[b]).wait()

        def scale(b):
            def edge(e, inner):
                g = e // 16
                lane = e - g * 16
                w16 = w_v[b, pl.ds(g * 16, 16)]
                wb = _broadcast_lane(w16, lane)
                for j in range(d_io // 16):
                    sl = pl.ds(j * 16, 16)
                    rows_v[b, e, sl] = rows_v[b, e, sl] * wb
                return inner
            lax.fori_loop(0, M, edge, 0)

        def zrow(i, carry):
            for j in range(d_io // 16):
                rows_v[0, i, pl.ds(j * 16, 16)] = jnp.zeros((16,), jnp.float32)
            return carry
        lax.fori_loop(0, M, zrow, 0)
        for q in range(zfull):
            pltpu.sync_copy(rows_v.at[0], acc_sh.at[pl.ds(row0 + q * M, M)])
        if zrem:
            pltpu.sync_copy(rows_v.at[0, pl.ds(0, zrem)],
                            acc_sh.at[pl.ds(row0 + zfull * M, zrem)])
        plsc.subcore_barrier()

        idx_fetch(0, 0)
        gather_start(0)

        def step(k, b):
            nb = 1 - b
            if b == 1:
                scatter_wait(nb)
            else:

                @pl.when(k >= 1)
                def _():
                    scatter_wait(nb)

            @pl.when(k + 1 < n_chunks)
            def _():
                idx_fetch(k + 1, nb)
                gather_start(nb)

            gather_wait(b)
            scale(b)
            scatter_start(b)

        def pair(k2, carry):
            step(k2 * 2, 0)
            step(k2 * 2 + 1, 1)
            return carry
        lax.fori_loop(0, n_chunks // 2, pair, 0)
        scatter_wait(1)
        plsc.subcore_barrier()

        pltpu.sync_copy(acc_sh.at[pl.ds(row0, rows_per_tile)],
                        out_hbm.at[c, pl.ds(row0, rows_per_tile)])

    return spmm


def _mm_body(x_ref, w_ref, o_ref):
    o_ref[...] = jnp.dot(x_ref[...], w_ref[...],
                         preferred_element_type=jnp.float32)


def _fuse1_body(p_ref, b_ref, w_ref, o_ref):
    h = jnp.maximum(p_ref[0] + p_ref[1] + b_ref[...], 0.0)
    o_ref[...] = jnp.dot(h, w_ref[...], preferred_element_type=jnp.float32)


def _fuse2_body(p_ref, b_ref, o_ref):
    s = p_ref[0] + p_ref[1] + b_ref[...]
    logits = s[:, :LABEL_DIM]
    m = jnp.max(logits, axis=1, keepdims=True)
    z = logits - m
    lse = jnp.log(jnp.sum(jnp.exp(z), axis=1, keepdims=True))
    o_ref[...] = z - lse


def kernel(x, edge_index, edge_weight, W1, b1, W2, b2):
    n = x.shape[0]
    e = edge_weight.shape[0]
    chunk_span = NC * NS * K * 2
    e_pad = ((e + chunk_span - 1) // chunk_span) * chunk_span
    row_span = NS * 8
    n_pad = ((n + row_span - 1) // row_span) * row_span

    src = edge_index[0].astype(jnp.int32)
    dst = edge_index[1].astype(jnp.int32)
    pad = e_pad - e
    if pad:
        src = jnp.pad(src, (0, pad))
        dst = jnp.pad(dst, (0, pad))
        edge_weight = jnp.pad(edge_weight, (0, pad))
    srcg = src
    dstg = dst
    wg = edge_weight

    w2p = jnp.pad(W2, ((0, 0), (0, LABEL_PAD - LABEL_DIM)))
    b1r = b1.reshape(1, HIDDEN)
    b2r = jnp.pad(b2, (0, LABEL_PAD - LABEL_DIM)).reshape(1, LABEL_PAD)

    h1 = pl.pallas_call(
        _mm_body,
        out_shape=jax.ShapeDtypeStruct((n, HIDDEN), jnp.float32),
    )(x, W1)

    spmm1 = _make_sc_spmm(n_pad, HIDDEN, e_pad, split=False, m=1)
    p1 = spmm1(srcg, dstg, wg, h1)

    h2 = pl.pallas_call(
        _fuse1_body,
        out_shape=jax.ShapeDtypeStruct((n_pad, LABEL_PAD), jnp.float32),
    )(p1, b1r, w2p)

    spmm2 = _make_sc_spmm(n_pad, LABEL_PAD, e_pad, split=False, m=1)
    p2 = spmm2(srcg, dstg, wg, h2)

    out = pl.pallas_call(
        _fuse2_body,
        out_shape=jax.ShapeDtypeStruct((n_pad, LABEL_DIM), jnp.float32),
    )(p2, b2r)
    return out[:n]


# speedup vs baseline: 1.4365x; 1.0004x over previous
import functools

import jax
import jax.numpy as jnp
from jax import lax
from jax.experimental import pallas as pl
from jax.experimental.pallas import tpu as pltpu
from jax.experimental.pallas import tpu_sc as plsc

N_NODES = 10000
D_IN = 128
HIDDEN = 128
LABEL_DIM = 40
LABEL_PAD = 48

NC = 2
NS = 16
K = 128


def _broadcast_lane(v16, lane):
    return lax.gather(
        v16, jnp.full((16, 1), lane, jnp.int32),
        lax.GatherDimensionNumbers(
            offset_dims=(), collapsed_slice_dims=(0,), start_index_map=(0,)),
        (1,), mode=lax.GatherScatterMode.PROMISE_IN_BOUNDS)


def _make_sc_spmm(n_pad, d, e_pad, split, m):
    d_io = d // 2 if split else d
    M = m * 128
    n_workers = NS if split else NC * NS
    per_w = e_pad // n_workers
    n_chunks = per_w // M
    assert n_chunks % 2 == 0 and per_w % M == 0
    rows_per_tile = n_pad // NS
    zfull, zrem = divmod(rows_per_tile, M)
    mesh = plsc.VectorSubcoreMesh(core_axis_name="c", subcore_axis_name="s")

    @functools.partial(
        pl.kernel,
        out_type=jax.ShapeDtypeStruct((NC, n_pad, d_io), jnp.float32),
        mesh=mesh,
        scratch_types=[
            pltpu.VMEM((2, M), jnp.int32),
            pltpu.VMEM((2, M), jnp.int32),
            pltpu.VMEM((2, M), jnp.float32),
            pltpu.VMEM((2, M, d_io), jnp.float32),
            pltpu.VMEM_SHARED((n_pad, d_io), jnp.float32),
            pltpu.SemaphoreType.DMA,
            pltpu.SemaphoreType.DMA,
            pltpu.SemaphoreType.DMA,
            pltpu.SemaphoreType.DMA,
        ],
        compiler_params=pltpu.CompilerParams(use_tc_tiling_on_sc=False),
    )
    def spmm(src_hbm, dst_hbm, w_hbm, h_hbm, out_hbm,
             sidx_v, didx_v, w_v, rows_v, acc_sh, g0, g1, s0, s1):
        c = lax.axis_index("c")
        s = lax.axis_index("s")
        wid = s if split else s * NC + c
        ebase = wid * n_chunks * M
        row0 = s * rows_per_tile
        gsem = (g0, g1)
        ssem = (s0, s1)

        def idx_fetch(k, b):
            pltpu.sync_copy(src_hbm.at[pl.ds(ebase + k * M, M)], sidx_v.at[b])
            pltpu.sync_copy(dst_hbm.at[pl.ds(ebase + k * M, M)], didx_v.at[b])
            pltpu.sync_copy(w_hbm.at[pl.ds(ebase + k * M, M)], w_v.at[b])
            if split:

                def sxf(gg, inner):
                    sl = pl.ds(gg * 16, 16)
                    sidx_v[b, sl] = sidx_v[b, sl] * 2 + c
                    return inner
                lax.fori_loop(0, M // 16, sxf, 0)

        def gather_start(b):
            pltpu.async_copy(h_hbm.at[sidx_v.at[b]], rows_v.at[b], gsem[b])

        def gather_wait(b):
            pltpu.make_async_copy(
                h_hbm.at[sidx_v.at[b]], rows_v.at[b], gsem[b]).wait()

        def scatter_start(b):
            pltpu.async_copy(
                rows_v.at[b], acc_sh.at[didx_v.at[b]], ssem[b], add=True)

        def scatter_wait(b):
            pltpu.make_async_copy(
                rows_v.at[b], acc_sh.at[didx_v.at[b]], ssem[b]).wait()

        def scale(b):
            def edge(e, inner):
                g = e // 16
                lane = e - g * 16
                w16 = w_v[b, pl.ds(g * 16, 16)]
                wb = _broadcast_lane(w16, lane)
                for j in range(d_io // 16):
                    sl = pl.ds(j * 16, 16)
                    rows_v[b, e, sl] = rows_v[b, e, sl] * wb
                return inner
            lax.fori_loop(0, M, edge, 0)

        def zrow(i, carry):
            for j in range(d_io // 16):
                rows_v[0, i, pl.ds(j * 16, 16)] = jnp.zeros((16,), jnp.float32)
            return carry
        lax.fori_loop(0, M, zrow, 0)
        for q in range(zfull):
            pltpu.sync_copy(rows_v.at[0], acc_sh.at[pl.ds(row0 + q * M, M)])
        if zrem:
            pltpu.sync_copy(rows_v.at[0, pl.ds(0, zrem)],
                            acc_sh.at[pl.ds(row0 + zfull * M, zrem)])
        plsc.subcore_barrier()

        idx_fetch(0, 0)
        gather_start(0)

        def step(k, b):
            nb = 1 - b
            if b == 1:
                scatter_wait(nb)
            else:

                @pl.when(k >= 1)
                def _():
                    scatter_wait(nb)

            @pl.when(k + 1 < n_chunks)
            def _():
                idx_fetch(k + 1, nb)
                gather_start(nb)

            gather_wait(b)
            scale(b)
            scatter_start(b)

        def pair(k2, carry):
            step(k2 * 2, 0)
            step(k2 * 2 + 1, 1)
            return carry
        lax.fori_loop(0, n_chunks // 2, pair, 0)
        scatter_wait(1)
        plsc.subcore_barrier()

        pltpu.sync_copy(acc_sh.at[pl.ds(row0, rows_per_tile)],
                        out_hbm.at[c, pl.ds(row0, rows_per_tile)])

    return spmm


def _mm_body(x_ref, w_ref, o_ref):
    o_ref[...] = jnp.dot(x_ref[...], w_ref[...],
                         preferred_element_type=jnp.float32)


def _fuse1_body(p_ref, b_ref, w_ref, o_ref):
    hcat = jnp.concatenate([p_ref[0], p_ref[1]], axis=1)
    h = jnp.maximum(hcat + b_ref[...], 0.0)
    o_ref[...] = jnp.dot(h, w_ref[...], preferred_element_type=jnp.float32)


def _fuse2_body(p_ref, b_ref, o_ref):
    s = p_ref[0] + p_ref[1] + b_ref[...]
    logits = s[:, :LABEL_DIM]
    m = jnp.max(logits, axis=1, keepdims=True)
    z = logits - m
    lse = jnp.log(jnp.sum(jnp.exp(z), axis=1, keepdims=True))
    o_ref[...] = z - lse


def kernel(x, edge_index, edge_weight, W1, b1, W2, b2):
    n = x.shape[0]
    e = edge_weight.shape[0]
    chunk_span = NC * NS * K * 2
    e_pad = ((e + chunk_span - 1) // chunk_span) * chunk_span
    row_span = NS * 8
    n_pad = ((n + row_span - 1) // row_span) * row_span

    src = edge_index[0].astype(jnp.int32)
    dst = edge_index[1].astype(jnp.int32)
    pad = e_pad - e
    if pad:
        src = jnp.pad(src, (0, pad))
        dst = jnp.pad(dst, (0, pad))
        edge_weight = jnp.pad(edge_weight, (0, pad))
    srcg = src
    dstg = dst
    wg = edge_weight

    w2p = jnp.pad(W2, ((0, 0), (0, LABEL_PAD - LABEL_DIM)))
    b1r = b1.reshape(1, HIDDEN)
    b2r = jnp.pad(b2, (0, LABEL_PAD - LABEL_DIM)).reshape(1, LABEL_PAD)

    h1 = pl.pallas_call(
        _mm_body,
        out_shape=jax.ShapeDtypeStruct((n, HIDDEN), jnp.float32),
    )(x, W1)

    spmm1 = _make_sc_spmm(n_pad, HIDDEN, e_pad, split=True, m=4)
    p1 = spmm1(srcg, dstg, wg, h1.reshape(2 * n, HIDDEN // 2))

    h2 = pl.pallas_call(
        _fuse1_body,
        out_shape=jax.ShapeDtypeStruct((n_pad, LABEL_PAD), jnp.float32),
    )(p1, b1r, w2p)

    spmm2 = _make_sc_spmm(n_pad, LABEL_PAD, e_pad, split=False, m=4)
    p2 = spmm2(srcg, dstg, wg, h2)

    out = pl.pallas_call(
        _fuse2_body,
        out_shape=jax.ShapeDtypeStruct((n_pad, LABEL_DIM), jnp.float32),
    )(p2, b2r)
    return out[:n]


# speedup vs baseline: 1.8319x; 1.2753x over previous
import functools

import jax
import jax.numpy as jnp
from jax import lax
from jax.experimental import pallas as pl
from jax.experimental.pallas import tpu as pltpu
from jax.experimental.pallas import tpu_sc as plsc

N_NODES = 10000
D_IN = 128
HIDDEN = 128
LABEL_DIM = 40
LABEL_PAD = 48

NC = 2
NS = 16
K = 128


def _broadcast_lane(v16, lane):
    return lax.gather(
        v16, jnp.full((16, 1), lane, jnp.int32),
        lax.GatherDimensionNumbers(
            offset_dims=(), collapsed_slice_dims=(0,), start_index_map=(0,)),
        (1,), mode=lax.GatherScatterMode.PROMISE_IN_BOUNDS)


def _make_sc_spmm(n_pad, d, e_pad, split, m, n_h):
    d_io = d // 2 if split else d
    M = m * 128
    n_workers = NS if split else NC * NS
    per_w = e_pad // n_workers
    n_chunks = per_w // M
    assert n_chunks % 2 == 0 and per_w % M == 0
    rows_per_tile = n_pad // NS
    zfull, zrem = divmod(rows_per_tile, M)
    mesh = plsc.VectorSubcoreMesh(core_axis_name="c", subcore_axis_name="s")

    @functools.partial(
        pl.kernel,
        out_type=jax.ShapeDtypeStruct((NC, n_pad, d_io), jnp.float32),
        mesh=mesh,
        scratch_types=[
            pltpu.VMEM((2, M), jnp.int32),
            pltpu.VMEM((2, M), jnp.int32),
            pltpu.VMEM((2, M), jnp.float32),
            pltpu.VMEM((2, M, d_io), jnp.float32),
            pltpu.VMEM_SHARED((n_pad, d_io), jnp.float32),
            pltpu.VMEM_SHARED((n_h, d_io), jnp.float32),
            pltpu.SemaphoreType.DMA,
            pltpu.SemaphoreType.DMA,
            pltpu.SemaphoreType.DMA,
            pltpu.SemaphoreType.DMA,
        ],
        compiler_params=pltpu.CompilerParams(use_tc_tiling_on_sc=False),
    )
    def spmm(src_hbm, dst_hbm, w_hbm, h_hbm, out_hbm,
             sidx_v, didx_v, w_v, rows_v, acc_sh, h_sh, g0, g1, s0, s1):
        c = lax.axis_index("c")
        s = lax.axis_index("s")
        wid = s if split else s * NC + c
        ebase = wid * n_chunks * M
        row0 = s * rows_per_tile
        gsem = (g0, g1)
        ssem = (s0, s1)

        def idx_fetch(k, b):
            pltpu.sync_copy(src_hbm.at[pl.ds(ebase + k * M, M)], sidx_v.at[b])
            pltpu.sync_copy(dst_hbm.at[pl.ds(ebase + k * M, M)], didx_v.at[b])
            pltpu.sync_copy(w_hbm.at[pl.ds(ebase + k * M, M)], w_v.at[b])

        def gather_start(b):
            pltpu.async_copy(h_sh.at[sidx_v.at[b]], rows_v.at[b], gsem[b])

        def gather_wait(b):
            pltpu.make_async_copy(
                h_sh.at[sidx_v.at[b]], rows_v.at[b], gsem[b]).wait()

        def scatter_start(b):
            pltpu.async_copy(
                rows_v.at[b], acc_sh.at[didx_v.at[b]], ssem[b], add=True)

        def scatter_wait(b):
            pltpu.make_async_copy(
                rows_v.at[b], acc_sh.at[didx_v.at[b]], ssem[b]).wait()

        def scale(b):
            def edge(e, inner):
                g = e // 16
                lane = e - g * 16
                w16 = w_v[b, pl.ds(g * 16, 16)]
                wb = _broadcast_lane(w16, lane)
                for j in range(d_io // 16):
                    sl = pl.ds(j * 16, 16)
                    rows_v[b, e, sl] = rows_v[b, e, sl] * wb
                return inner
            lax.fori_loop(0, M, edge, 0)

        def zrow(i, carry):
            for j in range(d_io // 16):
                rows_v[0, i, pl.ds(j * 16, 16)] = jnp.zeros((16,), jnp.float32)
            return carry
        lax.fori_loop(0, M, zrow, 0)
        for q in range(zfull):
            pltpu.sync_copy(rows_v.at[0], acc_sh.at[pl.ds(row0 + q * M, M)])
        if zrem:
            pltpu.sync_copy(rows_v.at[0, pl.ds(0, zrem)],
                            acc_sh.at[pl.ds(row0 + zfull * M, zrem)])
        hrows = n_h // NS
        hrow0 = s * hrows
        if split:
            pltpu.sync_copy(
                h_hbm.at[pl.ds(hrow0, hrows), pl.ds(c * d_io, d_io)],
                h_sh.at[pl.ds(hrow0, hrows)])
        else:
            pltpu.sync_copy(h_hbm.at[pl.ds(hrow0, hrows)],
                            h_sh.at[pl.ds(hrow0, hrows)])
        plsc.subcore_barrier()

        idx_fetch(0, 0)
        gather_start(0)

        def step(k, b):
            nb = 1 - b
            if b == 1:
                scatter_wait(nb)
            else:

                @pl.when(k >= 1)
                def _():
                    scatter_wait(nb)

            @pl.when(k + 1 < n_chunks)
            def _():
                idx_fetch(k + 1, nb)
                gather_start(nb)

            gather_wait(b)
            scale(b)
            scatter_start(b)

        def pair(k2, carry):
            step(k2 * 2, 0)
            step(k2 * 2 + 1, 1)
            return carry
        lax.fori_loop(0, n_chunks // 2, pair, 0)
        scatter_wait(1)
        plsc.subcore_barrier()

        pltpu.sync_copy(acc_sh.at[pl.ds(row0, rows_per_tile)],
                        out_hbm.at[c, pl.ds(row0, rows_per_tile)])

    return spmm


def _mm_body(x_ref, w_ref, o_ref):
    o_ref[...] = jnp.dot(x_ref[...], w_ref[...],
                         preferred_element_type=jnp.float32)


def _fuse1_body(p_ref, b_ref, w_ref, o_ref):
    hcat = jnp.concatenate([p_ref[0], p_ref[1]], axis=1)
    h = jnp.maximum(hcat + b_ref[...], 0.0)
    o_ref[...] = jnp.dot(h, w_ref[...], preferred_element_type=jnp.float32)


def _fuse2_body(p_ref, b_ref, o_ref):
    s = p_ref[0] + p_ref[1] + b_ref[...]
    logits = s[:, :LABEL_DIM]
    m = jnp.max(logits, axis=1, keepdims=True)
    z = logits - m
    lse = jnp.log(jnp.sum(jnp.exp(z), axis=1, keepdims=True))
    o_ref[...] = z - lse


def kernel(x, edge_index, edge_weight, W1, b1, W2, b2):
    n = x.shape[0]
    e = edge_weight.shape[0]
    chunk_span = NC * NS * K * 2
    e_pad = ((e + chunk_span - 1) // chunk_span) * chunk_span
    row_span = NS * 8
    n_pad = ((n + row_span - 1) // row_span) * row_span

    src = edge_index[0].astype(jnp.int32)
    dst = edge_index[1].astype(jnp.int32)
    pad = e_pad - e
    if pad:
        src = jnp.pad(src, (0, pad))
        dst = jnp.pad(dst, (0, pad))
        edge_weight = jnp.pad(edge_weight, (0, pad))
    srcg = src
    dstg = dst
    wg = edge_weight

    w2p = jnp.pad(W2, ((0, 0), (0, LABEL_PAD - LABEL_DIM)))
    b1r = b1.reshape(1, HIDDEN)
    b2r = jnp.pad(b2, (0, LABEL_PAD - LABEL_DIM)).reshape(1, LABEL_PAD)

    h1 = pl.pallas_call(
        _mm_body,
        out_shape=jax.ShapeDtypeStruct((n, HIDDEN), jnp.float32),
    )(x, W1)

    spmm1 = _make_sc_spmm(n_pad, HIDDEN, e_pad, split=True, m=2, n_h=n)
    p1 = spmm1(srcg, dstg, wg, h1)

    h2 = pl.pallas_call(
        _fuse1_body,
        out_shape=jax.ShapeDtypeStruct((n_pad, LABEL_PAD), jnp.float32),
    )(p1, b1r, w2p)

    spmm2 = _make_sc_spmm(n_pad, LABEL_PAD, e_pad, split=False, m=2, n_h=n_pad)
    p2 = spmm2(srcg, dstg, wg, h2)

    out = pl.pallas_call(
        _fuse2_body,
        out_shape=jax.ShapeDtypeStruct((n_pad, LABEL_DIM), jnp.float32),
    )(p2, b2r)
    return out[:n]


# speedup vs baseline: 1.9266x; 1.0517x over previous
import functools

import jax
import jax.numpy as jnp
from jax import lax
from jax.experimental import pallas as pl
from jax.experimental.pallas import tpu as pltpu
from jax.experimental.pallas import tpu_sc as plsc

N_NODES = 10000
D_IN = 128
HIDDEN = 128
LABEL_DIM = 40
LABEL_PAD = 48

NC = 2
NS = 16
K = 128


def _broadcast_lane(v16, lane):
    return lax.gather(
        v16, jnp.full((16, 1), lane, jnp.int32),
        lax.GatherDimensionNumbers(
            offset_dims=(), collapsed_slice_dims=(0,), start_index_map=(0,)),
        (1,), mode=lax.GatherScatterMode.PROMISE_IN_BOUNDS)


def _make_sc_spmm(n_pad, d, e_pad, split, m, n_h):
    d_io = d // 2 if split else d
    M = m * 128
    n_workers = NS if split else NC * NS
    per_w = e_pad // n_workers
    n_chunks = per_w // M
    assert n_chunks % 2 == 0 and per_w % M == 0
    rows_per_tile = n_pad // NS
    zfull, zrem = divmod(rows_per_tile, M)
    mesh = plsc.VectorSubcoreMesh(core_axis_name="c", subcore_axis_name="s")

    @functools.partial(
        pl.kernel,
        out_type=jax.ShapeDtypeStruct((NC, n_pad, d_io), jnp.float32),
        mesh=mesh,
        scratch_types=[
            pltpu.VMEM((2, M), jnp.int32),
            pltpu.VMEM((2, M), jnp.int32),
            pltpu.VMEM((2, M), jnp.float32),
            pltpu.VMEM((2, M, d_io), jnp.float32),
            pltpu.VMEM_SHARED((n_pad, d_io), jnp.float32),
            pltpu.VMEM_SHARED((n_h, d_io), jnp.float32),
            pltpu.SemaphoreType.DMA,
            pltpu.SemaphoreType.DMA,
            pltpu.SemaphoreType.DMA,
            pltpu.SemaphoreType.DMA,
        ],
        compiler_params=pltpu.CompilerParams(use_tc_tiling_on_sc=False),
    )
    def spmm(src_hbm, dst_hbm, w_hbm, h_hbm, out_hbm,
             sidx_v, didx_v, w_v, rows_v, acc_sh, h_sh, g0, g1, s0, s1):
        c = lax.axis_index("c")
        s = lax.axis_index("s")
        wid = s if split else s * NC + c
        ebase = wid * n_chunks * M
        row0 = s * rows_per_tile
        gsem = (g0, g1)
        ssem = (s0, s1)

        def idx_fetch(k, b):
            pltpu.sync_copy(src_hbm.at[pl.ds(ebase + k * M, M)], sidx_v.at[b])
            pltpu.sync_copy(dst_hbm.at[pl.ds(ebase + k * M, M)], didx_v.at[b])
            pltpu.sync_copy(w_hbm.at[pl.ds(ebase + k * M, M)], w_v.at[b])

        def gather_start(b):
            pltpu.async_copy(h_sh.at[sidx_v.at[b]], rows_v.at[b], gsem[b])

        def gather_wait(b):
            pltpu.make_async_copy(
                h_sh.at[sidx_v.at[b]], rows_v.at[b], gsem[b]).wait()

        def scatter_start(b):
            pltpu.async_copy(
                rows_v.at[b], acc_sh.at[didx_v.at[b]], ssem[b], add=True)

        def scatter_wait(b):
            pltpu.make_async_copy(
                rows_v.at[b], acc_sh.at[didx_v.at[b]], ssem[b]).wait()

        def scale(b):
            def grp(g, inner):
                base = g * 16
                w16 = w_v[b, pl.ds(base, 16)]
                for lane in range(16):
                    wb = _broadcast_lane(w16, lane)
                    for j in range(d_io // 16):
                        sl = pl.ds(j * 16, 16)
                        rows_v[b, base + lane, sl] = (
                            rows_v[b, base + lane, sl] * wb)
                return inner
            lax.fori_loop(0, M // 16, grp, 0)

        def zrow(i, carry):
            for j in range(d_io // 16):
                rows_v[0, i, pl.ds(j * 16, 16)] = jnp.zeros((16,), jnp.float32)
            return carry
        lax.fori_loop(0, M, zrow, 0)
        for q in range(zfull):
            pltpu.sync_copy(rows_v.at[0], acc_sh.at[pl.ds(row0 + q * M, M)])
        if zrem:
            pltpu.sync_copy(rows_v.at[0, pl.ds(0, zrem)],
                            acc_sh.at[pl.ds(row0 + zfull * M, zrem)])
        hrows = n_h // NS
        hrow0 = s * hrows
        if split:
            pltpu.sync_copy(
                h_hbm.at[pl.ds(hrow0, hrows), pl.ds(c * d_io, d_io)],
                h_sh.at[pl.ds(hrow0, hrows)])
        else:
            pltpu.sync_copy(h_hbm.at[pl.ds(hrow0, hrows)],
                            h_sh.at[pl.ds(hrow0, hrows)])
        plsc.subcore_barrier()

        idx_fetch(0, 0)
        gather_start(0)

        def step(k, b):
            nb = 1 - b
            if b == 1:
                scatter_wait(nb)
            else:

                @pl.when(k >= 1)
                def _():
                    scatter_wait(nb)

            @pl.when(k + 1 < n_chunks)
            def _():
                idx_fetch(k + 1, nb)
                gather_start(nb)

            gather_wait(b)
            scale(b)
            scatter_start(b)

        def pair(k2, carry):
            step(k2 * 2, 0)
            step(k2 * 2 + 1, 1)
            return carry
        lax.fori_loop(0, n_chunks // 2, pair, 0)
        scatter_wait(1)
        plsc.subcore_barrier()

        pltpu.sync_copy(acc_sh.at[pl.ds(row0, rows_per_tile)],
                        out_hbm.at[c, pl.ds(row0, rows_per_tile)])

    return spmm


def _mm_body(x_ref, w_ref, o_ref):
    o_ref[...] = jnp.dot(x_ref[...], w_ref[...],
                         preferred_element_type=jnp.float32)


def _fuse1_body(p_ref, b_ref, w_ref, o_ref):
    hcat = jnp.concatenate([p_ref[0], p_ref[1]], axis=1)
    h = jnp.maximum(hcat + b_ref[...], 0.0)
    o_ref[...] = jnp.dot(h, w_ref[...], preferred_element_type=jnp.float32)


def _fuse2_body(p_ref, b_ref, o_ref):
    s = p_ref[0] + p_ref[1] + b_ref[...]
    logits = s[:, :LABEL_DIM]
    m = jnp.max(logits, axis=1, keepdims=True)
    z = logits - m
    lse = jnp.log(jnp.sum(jnp.exp(z), axis=1, keepdims=True))
    o_ref[...] = z - lse


def kernel(x, edge_index, edge_weight, W1, b1, W2, b2):
    n = x.shape[0]
    e = edge_weight.shape[0]
    chunk_span = NC * NS * K * 2
    e_pad = ((e + chunk_span - 1) // chunk_span) * chunk_span
    row_span = NS * 8
    n_pad = ((n + row_span - 1) // row_span) * row_span

    src = edge_index[0].astype(jnp.int32)
    dst = edge_index[1].astype(jnp.int32)
    pad = e_pad - e
    if pad:
        src = jnp.pad(src, (0, pad))
        dst = jnp.pad(dst, (0, pad))
        edge_weight = jnp.pad(edge_weight, (0, pad))
    srcg = src
    dstg = dst
    wg = edge_weight

    w2p = jnp.pad(W2, ((0, 0), (0, LABEL_PAD - LABEL_DIM)))
    b1r = b1.reshape(1, HIDDEN)
    b2r = jnp.pad(b2, (0, LABEL_PAD - LABEL_DIM)).reshape(1, LABEL_PAD)

    h1 = pl.pallas_call(
        _mm_body,
        out_shape=jax.ShapeDtypeStruct((n, HIDDEN), jnp.float32),
    )(x, W1)

    spmm1 = _make_sc_spmm(n_pad, HIDDEN, e_pad, split=True, m=2, n_h=n)
    p1 = spmm1(srcg, dstg, wg, h1)

    h2 = pl.pallas_call(
        _fuse1_body,
        out_shape=jax.ShapeDtypeStruct((n_pad, LABEL_PAD), jnp.float32),
    )(p1, b1r, w2p)

    spmm2 = _make_sc_spmm(n_pad, LABEL_PAD, e_pad, split=False, m=2, n_h=n_pad)
    p2 = spmm2(srcg, dstg, wg, h2)

    out = pl.pallas_call(
        _fuse2_body,
        out_shape=jax.ShapeDtypeStruct((n_pad, LABEL_DIM), jnp.float32),
    )(p2, b2r)
    return out[:n]
